# Initial kernel scaffold; baseline (speedup 1.0000x reference)
import jax
import jax.numpy as jnp
from jax.experimental import pallas as pl


def kernel(x, edge_index, user_ids, current_roles, Wl1, bl1, Wr1, Wl2, bl2, Wr2, W_ih, W_hh, b_ih, b_hh, Wc1, bc1, Wc2, bc2, Wc3, bc3):
    raise NotImplementedError("write your pallas kernel here")


# speedup vs baseline: 5.9425x; 5.9425x over previous
import functools
import jax
import jax.numpy as jnp
from jax import lax
from jax.experimental import pallas as pl
from jax.experimental.pallas import tpu as pltpu
from jax.experimental.pallas import tpu_sc as plsc

N = 10000
E = 320000
D = 128
H = 128
B = 4096
R = 5

NC = 2
NS = 16
NW = NC * NS
CHUNK = 128
ROWS = E // CHUNK
NITER = (ROWS + NW - 1) // NW
RPT = 632
NP = NS * RPT
DA = 144

_mesh = functools.partial(
    plsc.VectorSubcoreMesh, core_axis_name="c", subcore_axis_name="s",
    num_cores=NC, num_subcores=NS)


def _sc_scatter(feats, src2d, dst2d, zeros_tile, F):
  @functools.partial(
      pl.kernel,
      out_type=jax.ShapeDtypeStruct((NC * NP, F), jnp.float32),
      mesh=_mesh(),
      scratch_types=[
          pltpu.VMEM((CHUNK,), jnp.int32),
          pltpu.VMEM((CHUNK,), jnp.int32),
          pltpu.VMEM((CHUNK, F), jnp.float32),
          pltpu.VMEM_SHARED((NP, F), jnp.float32),
          pltpu.SemaphoreType.DMA,
      ],
      compiler_params=pltpu.CompilerParams(use_tc_tiling_on_sc=False),
  )
  def k(feats_h, src_h, dst_h, zeros_h, out_h, src_v, dst_v, rows_v, acc, sem):
    c = lax.axis_index("c")
    s = lax.axis_index("s")
    wid = s * NC + c

    pltpu.sync_copy(zeros_h, acc.at[pl.ds(s * RPT, RPT)])
    plsc.subcore_barrier()

    def body(i, carry):
      r = wid + NW * i

      @pl.when(r < ROWS)
      def _():
        pltpu.sync_copy(src_h.at[r], src_v)
        pltpu.sync_copy(dst_h.at[r], dst_v)
        pltpu.async_copy(feats_h.at[src_v], rows_v, sem).wait()
        pltpu.sync_copy(rows_v, acc.at[dst_v], add=True)

      return carry

    lax.fori_loop(0, NITER, body, 0)
    plsc.subcore_barrier()

    pltpu.sync_copy(acc.at[pl.ds(s * RPT, RPT)],
                    out_h.at[pl.ds(c * NP + s * RPT, RPT)])

  return k(feats, src2d, dst2d, zeros_tile)


def _sc_gather(uids2d, s2a, s2b, h1, x, inv16):
  f32 = jnp.float32
  outs = (
      jax.ShapeDtypeStruct((B, 128), f32),
      jax.ShapeDtypeStruct((B, 128), f32),
      jax.ShapeDtypeStruct((B, 128), f32),
      jax.ShapeDtypeStruct((B, 128), f32),
      jax.ShapeDtypeStruct((B, 16), f32),
  )

  @functools.partial(
      pl.kernel,
      out_type=outs,
      mesh=_mesh(),
      scratch_types=[
          pltpu.VMEM((CHUNK,), jnp.int32),
          pltpu.VMEM((CHUNK, 128), jnp.float32),
          pltpu.VMEM((CHUNK, 16), jnp.float32),
          pltpu.SemaphoreType.DMA,
      ],
      compiler_params=pltpu.CompilerParams(use_tc_tiling_on_sc=False),
  )
  def k(uids_h, a_h, b_h, h1_h, x_h, inv_h,
        oa_h, ob_h, oh_h, ox_h, oi_h, uid_v, buf, buf16, sem):
    c = lax.axis_index("c")
    s = lax.axis_index("s")
    wid = s * NC + c
    pltpu.sync_copy(uids_h.at[wid], uid_v)
    for src_h, dst_h in ((a_h, oa_h), (b_h, ob_h), (h1_h, oh_h), (x_h, ox_h)):
      pltpu.async_copy(src_h.at[uid_v], buf, sem).wait()
      pltpu.sync_copy(buf, dst_h.at[pl.ds(wid * CHUNK, CHUNK)])
    pltpu.async_copy(inv_h.at[uid_v], buf16, sem).wait()
    pltpu.sync_copy(buf16, oi_h.at[pl.ds(wid * CHUNK, CHUNK)])

  return k(uids2d, s2a, s2b, h1, x, inv16)


def _tc_layer1(sa, sb, da, db, x, wl1t, bl1, wr1t):
  BLK = 1000
  f32 = jnp.float32

  def body(sa_r, sb_r, da_r, db_r, x_r, wl_r, bl_r, wr_r, h1_r, inv_r):
    s = sa_r[...] + sb_r[...]
    deg = jnp.maximum(da_r[...][:, 0:1] + db_r[...][:, 0:1], 1.0)
    inv = 1.0 / deg
    m = s * inv
    h = (jnp.dot(m, wl_r[...], preferred_element_type=f32) + bl_r[...]
         + jnp.dot(x_r[...], wr_r[...], preferred_element_type=f32))
    h1_r[...] = jnp.maximum(h, 0.0)
    inv_r[...] = jnp.broadcast_to(inv, (BLK, 16))

  blk = lambda m, n: pl.BlockSpec((m, n), lambda i: (i, 0))
  whole = lambda m, n: pl.BlockSpec((m, n), lambda i: (0, 0))
  return pl.pallas_call(
      body,
      grid=(N // BLK,),
      in_specs=[blk(BLK, 128), blk(BLK, 128), blk(BLK, 16), blk(BLK, 16),
                blk(BLK, 128), whole(128, 128), whole(1, 128),
                whole(128, 128)],
      out_specs=[blk(BLK, 128), blk(BLK, 16)],
      out_shape=[jax.ShapeDtypeStruct((N, 128), f32),
                 jax.ShapeDtypeStruct((N, 16), f32)],
  )(sa, sb, da, db, x, wl1t, bl1, wr1t)


def _tc_epilogue(ua, ub, uh1, ux, uinv, roh8,
                 wl2t, bl2, wr2t, wiha, wihb, wihr8, bih,
                 wc1at, wc1bt, bc1, wc2t, bc2, wc3t8, bc3p):
  BLK = 512
  f32 = jnp.float32

  def body(ua_r, ub_r, uh1_r, ux_r, uinv_r, roh_r,
           wl2_r, bl2_r, wr2_r, wiha_r, wihb_r, wihr_r, bih_r,
           wc1a_r, wc1b_r, bc1_r, wc2_r, bc2_r, wc3_r, bc3_r, out_r):
    dot = lambda a, b: jnp.dot(a, b, preferred_element_type=f32)
    m2 = (ua_r[...] + ub_r[...]) * uinv_r[...][:, 0:1]
    ue = dot(m2, wl2_r[...]) + bl2_r[...] + dot(uh1_r[...], wr2_r[...])
    ue = jnp.clip(ue, -10.0, 10.0)
    uf = jnp.clip(ux_r[...], -10.0, 10.0)
    gates = (dot(ue, wiha_r[...]) + dot(uf, wihb_r[...])
             + dot(roh_r[...], wihr_r[...]) + bih_r[...])
    i_g = gates[:, 0:128]
    g_g = gates[:, 256:384]
    o_g = gates[:, 384:512]
    cc = jax.nn.sigmoid(i_g) * jnp.tanh(g_g)
    lo = jnp.clip(jax.nn.sigmoid(o_g) * jnp.tanh(cc), -10.0, 10.0)
    z = jnp.maximum(dot(ue, wc1a_r[...]) + dot(lo, wc1b_r[...]) + bc1_r[...],
                    0.0)
    z2 = jnp.maximum(dot(z, wc2_r[...]) + bc2_r[...], 0.0)
    out_r[...] = dot(z2, wc3_r[...]) + bc3_r[...]

  blk = lambda m, n: pl.BlockSpec((m, n), lambda i: (i, 0))
  whole = lambda m, n: pl.BlockSpec((m, n), lambda i: (0, 0))
  return pl.pallas_call(
      body,
      grid=(B // BLK,),
      in_specs=[blk(BLK, 128), blk(BLK, 128), blk(BLK, 128), blk(BLK, 128),
                blk(BLK, 16), blk(BLK, 8),
                whole(128, 128), whole(1, 128), whole(128, 128),
                whole(128, 512), whole(128, 512), whole(8, 512),
                whole(1, 512),
                whole(128, 128), whole(128, 128), whole(1, 128),
                whole(128, 64), whole(1, 64), whole(64, 8), whole(1, 8)],
      out_specs=blk(BLK, 8),
      out_shape=jax.ShapeDtypeStruct((B, 8), f32),
  )(ua, ub, uh1, ux, uinv, roh8,
    wl2t, bl2, wr2t, wiha, wihb, wihr8, bih,
    wc1at, wc1bt, bc1, wc2t, bc2, wc3t8, bc3p)


def kernel(x, edge_index, user_ids, current_roles,
           Wl1, bl1, Wr1, Wl2, bl2, Wr2,
           W_ih, W_hh, b_ih, b_hh,
           Wc1, bc1, Wc2, bc2, Wc3, bc3):
  f32 = jnp.float32
  src2d = edge_index[0].reshape(ROWS, CHUNK)
  dst2d = edge_index[1].reshape(ROWS, CHUNK)
  uids2d = user_ids.reshape(NW, CHUNK)

  x_aug = jnp.concatenate(
      [x, jnp.ones((N, 1), f32), jnp.zeros((N, 15), f32)], axis=1)
  zeros_da = jnp.zeros((RPT, DA), f32)
  zeros_h = jnp.zeros((RPT, H), f32)

  s1 = _sc_scatter(x_aug, src2d, dst2d, zeros_da, DA)
  s1a, s1b = s1[:N], s1[NP:NP + N]

  h1, inv16 = _tc_layer1(
      s1a[:, :128], s1b[:, :128], s1a[:, 128:144], s1b[:, 128:144],
      x, Wl1.T, bl1.reshape(1, H), Wr1.T)

  s2 = _sc_scatter(h1, src2d, dst2d, zeros_h, H)
  s2a, s2b = s2[:N], s2[NP:NP + N]

  ua, ub, uh1, ux, uinv = _sc_gather(uids2d, s2a, s2b, h1, x, inv16)

  roh8 = jax.nn.one_hot(current_roles, 8, dtype=f32)
  wihr8 = jnp.zeros((8, 4 * H), f32).at[:R].set(W_ih[:, 2 * H:].T)
  wc3t8 = jnp.zeros((H // 2, 8), f32).at[:, :R].set(Wc3.T)
  bc3p = jnp.zeros((1, 8), f32).at[:, :R].set(bc3)

  out8 = _tc_epilogue(
      ua, ub, uh1, ux, uinv, roh8,
      Wl2.T, bl2.reshape(1, H), Wr2.T,
      W_ih[:, :H].T, W_ih[:, H:2 * H].T, wihr8,
      (b_ih + b_hh).reshape(1, 4 * H),
      Wc1[:, :H].T, Wc1[:, H:].T, bc1.reshape(1, H),
      Wc2.T, bc2.reshape(1, H // 2), wc3t8, bc3p)
  return out8[:, :R]
